# baseline (speedup 1.0000x reference)
import functools

import jax
import jax.numpy as jnp
from jax import lax
from jax.experimental import pallas as pl
from jax.experimental.pallas import tpu as pltpu
from jax.experimental.pallas import tpu_sc as plsc

_N = 10000
_E = 320000
_G = 32
_D = 128

_NCORES = 2
_NSUB = 16
_DH = _D // 2
_EPT = _E // _NSUB
_BATCH = 128
_NBAT = (_EPT + _BATCH - 1) // _BATCH
_PAIRS = (_NBAT - 1) // 2
_NPAD = _N + 8
_RPT = 632
_RLAST = _N - (_NSUB - 1) * _RPT
_ZLAST = _NPAD - (_NSUB - 1) * _RPT


def _make_seg_sum(n, d):
    mesh = plsc.VectorSubcoreMesh(core_axis_name="c", subcore_axis_name="s")
    scratch = [
        pltpu.VMEM((_NBAT, _BATCH), jnp.int32),
        pltpu.VMEM((_NBAT, _BATCH), jnp.int32),
        pltpu.VMEM((_BATCH, d), jnp.float32),
        pltpu.VMEM((_BATCH, d), jnp.float32),
        pltpu.VMEM_SHARED((_NPAD, d), jnp.float32),
        pltpu.SemaphoreType.DMA,
        pltpu.SemaphoreType.DMA,
    ]

    @functools.partial(
        pl.kernel,
        out_type=jax.ShapeDtypeStruct((n, 2, d), jnp.float32),
        mesh=mesh,
        scratch_types=scratch,
        compiler_params=pltpu.CompilerParams(use_tc_tiling_on_sc=False),
    )
    def seg(table, srcs, dsts, zeros, out,
            src_t, dst_t, rows0, rows1, acc, sem0, sem1):
        c = lax.axis_index("c")
        s = lax.axis_index("s")
        r0 = s * _RPT
        rows = (rows0, rows1)
        sems = (sem0, sem1)

        pltpu.sync_copy(srcs.at[c * _NSUB + s], src_t)
        pltpu.sync_copy(dsts.at[s], dst_t)

        pltpu.sync_copy(zeros, rows0)
        for k in range(4):
            pltpu.sync_copy(rows0, acc.at[pl.ds(r0 + 128 * k, 128)])

        @pl.when(s < _NSUB - 1)
        def _():
            pltpu.sync_copy(rows0.at[pl.ds(0, 120)],
                            acc.at[pl.ds(r0 + 512, 120)])

        @pl.when(s == _NSUB - 1)
        def _():
            pltpu.sync_copy(rows0.at[pl.ds(0, 16)],
                            acc.at[pl.ds(r0 + 512, 16)])

        plsc.subcore_barrier()

        def gstart(k, slot):
            pltpu.async_copy(table.at[src_t.at[k]], rows[slot], sems[slot])

        def gwait(slot):
            pltpu.make_async_copy(table.at[src_t.at[0]], rows[slot],
                                  sems[slot]).wait()

        def scat(k, slot):
            pltpu.sync_copy(rows[slot], acc.at[dst_t.at[k]], add=True)

        gstart(0, 0)

        def body(j, carry):
            k = 2 * j
            gstart(k + 1, 1)
            gwait(0)
            scat(k, 0)
            gstart(k + 2, 0)
            gwait(1)
            scat(k + 1, 1)
            return carry

        lax.fori_loop(0, _PAIRS, body, 0)
        gwait(0)
        scat(_NBAT - 1, 0)

        plsc.subcore_barrier()

        def chunk_out(off, size, buf):
            pltpu.sync_copy(acc.at[pl.ds(r0 + off, size)],
                            buf.at[pl.ds(0, size)])
            pltpu.sync_copy(buf.at[pl.ds(0, size)],
                            out.at[pl.ds(r0 + off, size), c])

        for k in range(4):
            chunk_out(128 * k, 128, rows[k % 2])

        @pl.when(s < _NSUB - 1)
        def _():
            chunk_out(512, 120, rows0)

        @pl.when(s == _NSUB - 1)
        def _():
            chunk_out(512, 8, rows0)

    return seg


_SEG_SUM_CACHE = []


def _seg_sum(table_split, srcs, dsts, zeros):
    if not _SEG_SUM_CACHE:
        _SEG_SUM_CACHE.append(_make_seg_sum(_N, _DH))
    return _SEG_SUM_CACHE[0](table_split, srcs, dsts, zeros)


def _prep_edges(src, dst):
    s2 = jnp.pad(2 * src.reshape(_NSUB, _EPT),
                 ((0, 0), (0, _NBAT * _BATCH - _EPT)))
    srcs = jnp.concatenate([s2, s2 + 1], axis=0).reshape(
        2 * _NSUB, _NBAT, _BATCH)
    d2 = jnp.pad(dst.reshape(_NSUB, _EPT),
                 ((0, 0), (0, _NBAT * _BATCH - _EPT)),
                 constant_values=_N)
    dsts = d2.reshape(_NSUB, _NBAT, _BATCH)
    return srcs, dsts


def _split(a):
    return a.reshape(2 * _N, _DH)


_BR = 1000
_NBLK = _N // _BR


def _elu(y):
    return jnp.where(y > 0, y, jnp.exp(jnp.minimum(y, 0.0)) - 1.0)


def _lbe_body(has_agg, f):
    def body(*refs):
        if has_agg:
            (x_ref, p_ref, w_ref, b_ref, g_ref, bb_ref,
             z_ref, y_acc, s1, s2) = refs
        else:
            (x_ref, w_ref, b_ref, g_ref, bb_ref, z_ref, y_acc, s1, s2) = refs
        p = pl.program_id(0)
        i = pl.program_id(1)

        @pl.when(p == 0)
        def _():
            t = x_ref[...]
            if has_agg:
                t = t + p_ref[...]
            y = jnp.dot(t, w_ref[...], preferred_element_type=jnp.float32) \
                + b_ref[...]
            y_acc[pl.ds(i * _BR, _BR), :] = y
            i1 = jnp.sum(y, axis=0, keepdims=True)
            i2 = jnp.sum(y * y, axis=0, keepdims=True)
            s1[0:1, :] = jnp.where(i == 0, i1, s1[0:1, :] + i1)
            s2[0:1, :] = jnp.where(i == 0, i2, s2[0:1, :] + i2)

        @pl.when(p == 1)
        def _():
            y = y_acc[pl.ds(i * _BR, _BR), :]
            m = s1[0:1, :] * (1.0 / _N)
            v = s2[0:1, :] * (1.0 / _N) - m * m
            z_ref[...] = _elu((y - m) * lax.rsqrt(v + 1e-5) * g_ref[...]
                              + bb_ref[...])

    return body


def _lbe(x, w, b, g, bb, p=None):
    k = x.shape[1]
    f = w.shape[1]
    has_agg = p is not None
    row = lambda pp, ii: (ii, 0)
    in_specs = [pl.BlockSpec((_BR, k), row)]
    args = [x]
    if has_agg:
        in_specs += [pl.BlockSpec((_BR, k), row)]
        args += [p]
    in_specs += [pl.BlockSpec((k, f), lambda pp, ii: (0, 0))] + \
        [pl.BlockSpec((1, f), lambda pp, ii: (0, 0))] * 3
    args += [w, b.reshape(1, -1), g.reshape(1, -1), bb.reshape(1, -1)]
    return pl.pallas_call(
        _lbe_body(has_agg, f),
        grid=(2, _NBLK),
        in_specs=in_specs,
        out_specs=pl.BlockSpec((_BR, f), row),
        out_shape=jax.ShapeDtypeStruct((_N, f), jnp.float32),
        scratch_shapes=[pltpu.VMEM((_N, f), jnp.float32),
                        pltpu.VMEM((8, f), jnp.float32),
                        pltpu.VMEM((8, f), jnp.float32)],
    )(*args)


def _tc_pool(z0_ref, z1_ref, z2_ref, lws_ref, batch_ref, out_ref, zz_ref, acc):
    i = pl.program_id(0)
    lws = lws_ref[...]
    z0 = z0_ref[...] * lws[0, 0]
    z1 = z1_ref[...] * lws[0, 1]
    z2 = z2_ref[...] * lws[0, 2]
    zz_ref[...] = z0 + z1 + z2
    big = jnp.concatenate([z0, z1, z2], axis=1)
    batch = batch_ref[...]
    ninf = jnp.float32(-jnp.inf)
    rows = []
    for g in range(_G):
        rows.append(jnp.max(jnp.where(batch == g, big, ninf), axis=0))
    blockmax = jnp.stack(rows)
    prev = jnp.where(i == 0, jnp.full((_G, 30), ninf), acc[...])
    acc[...] = jnp.maximum(prev, blockmax)

    @pl.when(i == _NBLK - 1)
    def _():
        oc = acc[...]
        out_ref[...] = oc[:, 0:10] + oc[:, 10:20] + oc[:, 20:30]


def _f32(*shape):
    return jax.ShapeDtypeStruct(shape, jnp.float32)


def kernel(x, edge_index, batch, lw,
           lin0_W, lin0_b, bn0_g, bn0_b,
           lin1_W, lin1_b, bn1_g, bn1_b,
           lin2_W, lin2_b, bn2_g, bn2_b,
           conv0_W, conv0_b, cbn0_g, cbn0_b,
           conv1_W, conv1_b, cbn1_g, cbn1_b):
    srcs, dsts = _prep_edges(edge_index[0], edge_index[1])
    zeros = jnp.zeros((_BATCH, _DH), jnp.float32)

    p1 = _seg_sum(_split(x), srcs, dsts, zeros).reshape(_N, _D)

    z0 = _lbe(x, lin0_W, lin0_b, bn0_g, bn0_b)

    h1 = _lbe(x, conv0_W, conv0_b, cbn0_g, cbn0_b, p=p1)
    z1 = _lbe(h1, lin1_W, lin1_b, bn1_g, bn1_b)

    p2 = _seg_sum(_split(h1), srcs, dsts, zeros).reshape(_N, _D)

    h2 = _lbe(h1, conv1_W, conv1_b, cbn1_g, cbn1_b, p=p2)
    z2 = _lbe(h2, lin2_W, lin2_b, bn2_g, bn2_b)

    rowspec = pl.BlockSpec((_BR, 10), lambda i: (i, 0))
    whole = lambda shp: pl.BlockSpec(shp, lambda i: (0, 0))
    out, zz = pl.pallas_call(
        _tc_pool,
        grid=(_NBLK,),
        in_specs=[rowspec, rowspec, rowspec, whole((1, 3)),
                  pl.BlockSpec((_BR, 1), lambda i: (i, 0))],
        out_specs=[whole((_G, 10)), rowspec],
        out_shape=[_f32(_G, 10), _f32(_N, 10)],
        scratch_shapes=[pltpu.VMEM((_G, 30), jnp.float32)],
    )(z0, z1, z2, lw.reshape(1, 3), batch.reshape(-1, 1))
    return (out, zz, h2)


# speedup vs baseline: 1.1767x; 1.1767x over previous
import functools

import jax
import jax.numpy as jnp
from jax import lax
from jax.experimental import pallas as pl
from jax.experimental.pallas import tpu as pltpu
from jax.experimental.pallas import tpu_sc as plsc

_N = 10000
_E = 320000
_G = 32
_D = 128

_NCORES = 2
_NSUB = 16
_DH = _D // 2
_EPT = _E // _NSUB
_BATCH = 128
_NBAT = (_EPT + _BATCH - 1) // _BATCH
_PAIRS = (_NBAT - 1) // 2
_NPAD = _N + 8
_RPT = 632
_RLAST = _N - (_NSUB - 1) * _RPT
_ZLAST = _NPAD - (_NSUB - 1) * _RPT


def _make_seg_sum(n, d):
    mesh = plsc.VectorSubcoreMesh(core_axis_name="c", subcore_axis_name="s")
    scratch = [
        pltpu.VMEM((_NBAT, _BATCH), jnp.int32),
        pltpu.VMEM((_NBAT, _BATCH), jnp.int32),
        pltpu.VMEM((_BATCH, d), jnp.float32),
        pltpu.VMEM((_BATCH, d), jnp.float32),
        pltpu.VMEM_SHARED((_NPAD, d), jnp.float32),
        pltpu.SemaphoreType.DMA,
        pltpu.SemaphoreType.DMA,
    ]

    @functools.partial(
        pl.kernel,
        out_type=jax.ShapeDtypeStruct((2 * n, d), jnp.float32),
        mesh=mesh,
        scratch_types=scratch,
        compiler_params=pltpu.CompilerParams(use_tc_tiling_on_sc=False),
    )
    def seg(table, srcs, dsts, zeros, out,
            src_t, dst_t, rows0, rows1, acc, sem0, sem1):
        c = lax.axis_index("c")
        s = lax.axis_index("s")
        r0 = s * _RPT
        rows = (rows0, rows1)
        sems = (sem0, sem1)

        pltpu.sync_copy(srcs.at[c * _NSUB + s], src_t)
        pltpu.sync_copy(dsts.at[s], dst_t)

        pltpu.sync_copy(zeros, rows0)
        for k in range(4):
            pltpu.sync_copy(rows0, acc.at[pl.ds(r0 + 128 * k, 128)])

        @pl.when(s < _NSUB - 1)
        def _():
            pltpu.sync_copy(rows0.at[pl.ds(0, 120)],
                            acc.at[pl.ds(r0 + 512, 120)])

        @pl.when(s == _NSUB - 1)
        def _():
            pltpu.sync_copy(rows0.at[pl.ds(0, 16)],
                            acc.at[pl.ds(r0 + 512, 16)])

        plsc.subcore_barrier()

        def gstart(k, slot):
            pltpu.async_copy(table.at[src_t.at[k]], rows[slot], sems[slot])

        def gwait(slot):
            pltpu.make_async_copy(table.at[src_t.at[0]], rows[slot],
                                  sems[slot]).wait()

        def scat(k, slot):
            pltpu.sync_copy(rows[slot], acc.at[dst_t.at[k]], add=True)

        gstart(0, 0)

        def body(j, carry):
            k = 2 * j
            gstart(k + 1, 1)
            gwait(0)
            scat(k, 0)
            gstart(k + 2, 0)
            gwait(1)
            scat(k + 1, 1)
            return carry

        lax.fori_loop(0, _PAIRS, body, 0)
        gwait(0)
        scat(_NBAT - 1, 0)

        plsc.subcore_barrier()

        def chunk_out(off, size, buf):
            pltpu.sync_copy(acc.at[pl.ds(r0 + off, size)],
                            buf.at[pl.ds(0, size)])
            pltpu.sync_copy(buf.at[pl.ds(0, size)],
                            out.at[pl.ds(c * n + r0 + off, size)])

        for k in range(4):
            chunk_out(128 * k, 128, rows[k % 2])

        @pl.when(s < _NSUB - 1)
        def _():
            chunk_out(512, 120, rows0)

        @pl.when(s == _NSUB - 1)
        def _():
            chunk_out(512, 8, rows0)

    return seg


_SEG_SUM_CACHE = []


def _seg_sum(table_split, srcs, dsts, zeros):
    if not _SEG_SUM_CACHE:
        _SEG_SUM_CACHE.append(_make_seg_sum(_N, _DH))
    return _SEG_SUM_CACHE[0](table_split, srcs, dsts, zeros)


def _prep_edges(src, dst):
    s2 = jnp.pad(src.reshape(_NSUB, _EPT),
                 ((0, 0), (0, _NBAT * _BATCH - _EPT)))
    srcs = jnp.concatenate([s2, s2 + _N], axis=0).reshape(
        2 * _NSUB, _NBAT, _BATCH)
    d2 = jnp.pad(dst.reshape(_NSUB, _EPT),
                 ((0, 0), (0, _NBAT * _BATCH - _EPT)),
                 constant_values=_N)
    dsts = d2.reshape(_NSUB, _NBAT, _BATCH)
    return srcs, dsts


def _split(a):
    return jnp.concatenate([a[:, :_DH], a[:, _DH:]], axis=0)


_BR = 1000
_NBLK = _N // _BR


def _elu(y):
    return jnp.where(y > 0, y, jnp.exp(jnp.minimum(y, 0.0)) - 1.0)


def _lbe_body(has_agg, f):
    def body(*refs):
        if has_agg:
            (x_ref, pa_ref, pb_ref, w_ref, b_ref, g_ref, bb_ref,
             z_ref, y_acc, s1, s2) = refs
        else:
            (x_ref, w_ref, b_ref, g_ref, bb_ref, z_ref, y_acc, s1, s2) = refs
        p = pl.program_id(0)
        i = pl.program_id(1)

        @pl.when(p == 0)
        def _():
            t = x_ref[...]
            if has_agg:
                t = t + jnp.concatenate([pa_ref[...], pb_ref[...]], axis=1)
            y = jnp.dot(t, w_ref[...], preferred_element_type=jnp.float32) \
                + b_ref[...]
            y_acc[pl.ds(i * _BR, _BR), :] = y
            i1 = jnp.sum(y, axis=0, keepdims=True)
            i2 = jnp.sum(y * y, axis=0, keepdims=True)
            s1[0:1, :] = jnp.where(i == 0, i1, s1[0:1, :] + i1)
            s2[0:1, :] = jnp.where(i == 0, i2, s2[0:1, :] + i2)

        @pl.when(p == 1)
        def _():
            y = y_acc[pl.ds(i * _BR, _BR), :]
            m = s1[0:1, :] * (1.0 / _N)
            v = s2[0:1, :] * (1.0 / _N) - m * m
            z_ref[...] = _elu((y - m) * lax.rsqrt(v + 1e-5) * g_ref[...]
                              + bb_ref[...])

    return body


def _lbe(x, w, b, g, bb, p=None):
    k = x.shape[1]
    f = w.shape[1]
    has_agg = p is not None
    row = lambda pp, ii: (ii, 0)
    in_specs = [pl.BlockSpec((_BR, k), row)]
    args = [x]
    if has_agg:
        in_specs += [pl.BlockSpec((_BR, _DH), row),
                     pl.BlockSpec((_BR, _DH), lambda pp, ii: (ii + _NBLK, 0))]
        args += [p, p]
    in_specs += [pl.BlockSpec((k, f), lambda pp, ii: (0, 0))] + \
        [pl.BlockSpec((1, f), lambda pp, ii: (0, 0))] * 3
    args += [w, b.reshape(1, -1), g.reshape(1, -1), bb.reshape(1, -1)]
    return pl.pallas_call(
        _lbe_body(has_agg, f),
        grid=(2, _NBLK),
        in_specs=in_specs,
        out_specs=pl.BlockSpec((_BR, f), row),
        out_shape=jax.ShapeDtypeStruct((_N, f), jnp.float32),
        scratch_shapes=[pltpu.VMEM((_N, f), jnp.float32),
                        pltpu.VMEM((8, f), jnp.float32),
                        pltpu.VMEM((8, f), jnp.float32)],
    )(*args)


def _gin_body(f1, f2):
    def body(x_ref, pa_ref, pb_ref, cw_ref, cb_ref, cg_ref, cbb_ref,
             lw_ref, lb_ref, lg_ref, lbb_ref, h_ref, z_ref,
             yc_acc, y1_acc, c1, c2, l1, l2):
        p = pl.program_id(0)
        i = pl.program_id(1)
        rows = pl.ds(i * _BR, _BR)

        @pl.when(p == 0)
        def _():
            t = x_ref[...] + jnp.concatenate([pa_ref[...], pb_ref[...]],
                                             axis=1)
            yc = jnp.dot(t, cw_ref[...], preferred_element_type=jnp.float32) \
                + cb_ref[...]
            yc_acc[rows, :] = yc
            i1 = jnp.sum(yc, axis=0, keepdims=True)
            i2 = jnp.sum(yc * yc, axis=0, keepdims=True)
            c1[0:1, :] = jnp.where(i == 0, i1, c1[0:1, :] + i1)
            c2[0:1, :] = jnp.where(i == 0, i2, c2[0:1, :] + i2)

        @pl.when(p == 1)
        def _():
            yc = yc_acc[rows, :]
            m = c1[0:1, :] * (1.0 / _N)
            v = c2[0:1, :] * (1.0 / _N) - m * m
            h = _elu((yc - m) * lax.rsqrt(v + 1e-5) * cg_ref[...]
                     + cbb_ref[...])
            yc_acc[rows, :] = h
            y1 = jnp.dot(h, lw_ref[...], preferred_element_type=jnp.float32) \
                + lb_ref[...]
            y1_acc[rows, :] = y1
            i1 = jnp.sum(y1, axis=0, keepdims=True)
            i2 = jnp.sum(y1 * y1, axis=0, keepdims=True)
            l1[0:1, :] = jnp.where(i == 0, i1, l1[0:1, :] + i1)
            l2[0:1, :] = jnp.where(i == 0, i2, l2[0:1, :] + i2)

        @pl.when(p == 2)
        def _():
            h_ref[...] = yc_acc[rows, :]
            y1 = y1_acc[rows, :]
            m = l1[0:1, :] * (1.0 / _N)
            v = l2[0:1, :] * (1.0 / _N) - m * m
            z_ref[...] = _elu((y1 - m) * lax.rsqrt(v + 1e-5) * lg_ref[...]
                              + lbb_ref[...])

    return body


def _gin_block(x, p, cw, cb, cg, cbb, lw, lb, lg, lbb):
    k = x.shape[1]
    f1 = cw.shape[1]
    f2 = lw.shape[1]
    ph0 = lambda pp, ii: (jnp.where(pp == 0, ii, 0), 0)
    ph2 = lambda pp, ii: (jnp.where(pp == 2, ii, 0), 0)
    fixed = lambda pp, ii: (0, 0)
    r2 = lambda a: a.reshape(1, -1)
    in_specs = [
        pl.BlockSpec((_BR, k), ph0),
        pl.BlockSpec((_BR, _DH), ph0),
        pl.BlockSpec((_BR, _DH), lambda pp, ii: (jnp.where(pp == 0, ii, 0)
                                                 + _NBLK, 0)),
        pl.BlockSpec((k, f1), fixed), pl.BlockSpec((1, f1), fixed),
        pl.BlockSpec((1, f1), fixed), pl.BlockSpec((1, f1), fixed),
        pl.BlockSpec((f1, f2), fixed), pl.BlockSpec((1, f2), fixed),
        pl.BlockSpec((1, f2), fixed), pl.BlockSpec((1, f2), fixed),
    ]
    return pl.pallas_call(
        _gin_body(f1, f2),
        grid=(3, _NBLK),
        in_specs=in_specs,
        out_specs=[pl.BlockSpec((_BR, f1), ph2), pl.BlockSpec((_BR, f2), ph2)],
        out_shape=[jax.ShapeDtypeStruct((_N, f1), jnp.float32),
                   jax.ShapeDtypeStruct((_N, f2), jnp.float32)],
        scratch_shapes=[pltpu.VMEM((_N, f1), jnp.float32),
                        pltpu.VMEM((_N, f2), jnp.float32),
                        pltpu.VMEM((8, f1), jnp.float32),
                        pltpu.VMEM((8, f1), jnp.float32),
                        pltpu.VMEM((8, f2), jnp.float32),
                        pltpu.VMEM((8, f2), jnp.float32)],
    )(x, p, p, cw, r2(cb), r2(cg), r2(cbb), lw, r2(lb), r2(lg), r2(lbb))


def _tc_pool(z0_ref, z1_ref, z2_ref, lws_ref, batch_ref, out_ref, zz_ref, acc):
    i = pl.program_id(0)
    lws = lws_ref[...]
    z0 = z0_ref[...] * lws[0, 0]
    z1 = z1_ref[...] * lws[0, 1]
    z2 = z2_ref[...] * lws[0, 2]
    zz_ref[...] = z0 + z1 + z2
    big = jnp.concatenate([z0, z1, z2], axis=1)
    batch = batch_ref[...]
    ninf = jnp.float32(-jnp.inf)
    rows = []
    for g in range(_G):
        rows.append(jnp.max(jnp.where(batch == g, big, ninf), axis=0))
    blockmax = jnp.stack(rows)
    prev = jnp.where(i == 0, jnp.full((_G, 30), ninf), acc[...])
    acc[...] = jnp.maximum(prev, blockmax)

    @pl.when(i == _NBLK - 1)
    def _():
        oc = acc[...]
        out_ref[...] = oc[:, 0:10] + oc[:, 10:20] + oc[:, 20:30]


def _f32(*shape):
    return jax.ShapeDtypeStruct(shape, jnp.float32)


def kernel(x, edge_index, batch, lw,
           lin0_W, lin0_b, bn0_g, bn0_b,
           lin1_W, lin1_b, bn1_g, bn1_b,
           lin2_W, lin2_b, bn2_g, bn2_b,
           conv0_W, conv0_b, cbn0_g, cbn0_b,
           conv1_W, conv1_b, cbn1_g, cbn1_b):
    srcs, dsts = _prep_edges(edge_index[0], edge_index[1])
    zeros = jnp.zeros((_BATCH, _DH), jnp.float32)

    p1 = _seg_sum(_split(x), srcs, dsts, zeros)

    z0 = _lbe(x, lin0_W, lin0_b, bn0_g, bn0_b)

    h1, z1 = _gin_block(x, p1, conv0_W, conv0_b, cbn0_g, cbn0_b,
                        lin1_W, lin1_b, bn1_g, bn1_b)

    p2 = _seg_sum(_split(h1), srcs, dsts, zeros)

    h2, z2 = _gin_block(h1, p2, conv1_W, conv1_b, cbn1_g, cbn1_b,
                        lin2_W, lin2_b, bn2_g, bn2_b)

    rowspec = pl.BlockSpec((_BR, 10), lambda i: (i, 0))
    whole = lambda shp: pl.BlockSpec(shp, lambda i: (0, 0))
    out, zz = pl.pallas_call(
        _tc_pool,
        grid=(_NBLK,),
        in_specs=[rowspec, rowspec, rowspec, whole((1, 3)),
                  pl.BlockSpec((_BR, 1), lambda i: (i, 0))],
        out_specs=[whole((_G, 10)), rowspec],
        out_shape=[_f32(_G, 10), _f32(_N, 10)],
        scratch_shapes=[pltpu.VMEM((_G, 30), jnp.float32)],
    )(z0, z1, z2, lw.reshape(1, 3), batch.reshape(-1, 1))
    return (out, zz, h2)


# speedup vs baseline: 1.1775x; 1.0007x over previous
import functools

import jax
import jax.numpy as jnp
from jax import lax
from jax.experimental import pallas as pl
from jax.experimental.pallas import tpu as pltpu
from jax.experimental.pallas import tpu_sc as plsc

_N = 10000
_E = 320000
_G = 32
_D = 128

_NCORES = 2
_NSUB = 16
_DH = _D // 2
_EPT = _E // _NSUB
_BATCH = 128
_NBAT = (_EPT + _BATCH - 1) // _BATCH
_PAIRS = (_NBAT - 1) // 2
_NPAD = _N + 8
_RPT = 632
_RLAST = _N - (_NSUB - 1) * _RPT
_ZLAST = _NPAD - (_NSUB - 1) * _RPT


def _make_seg_sum(n, d):
    mesh = plsc.VectorSubcoreMesh(core_axis_name="c", subcore_axis_name="s")
    scratch = [
        pltpu.VMEM((_NBAT, _BATCH), jnp.int32),
        pltpu.VMEM((_NBAT, _BATCH), jnp.int32),
        pltpu.VMEM((_BATCH, d), jnp.float32),
        pltpu.VMEM((_BATCH, d), jnp.float32),
        pltpu.VMEM_SHARED((_NPAD, d), jnp.float32),
        pltpu.SemaphoreType.DMA,
        pltpu.SemaphoreType.DMA,
    ]

    @functools.partial(
        pl.kernel,
        out_type=jax.ShapeDtypeStruct((2 * n, d), jnp.float32),
        mesh=mesh,
        scratch_types=scratch,
        compiler_params=pltpu.CompilerParams(use_tc_tiling_on_sc=False),
    )
    def seg(table, srcs, dsts, zeros, out,
            src_t, dst_t, rows0, rows1, acc, sem0, sem1):
        c = lax.axis_index("c")
        s = lax.axis_index("s")
        r0 = s * _RPT
        rows = (rows0, rows1)
        sems = (sem0, sem1)

        pltpu.sync_copy(srcs.at[c * _NSUB + s], src_t)
        pltpu.sync_copy(dsts.at[s], dst_t)

        pltpu.sync_copy(zeros, rows0)
        for k in range(4):
            pltpu.sync_copy(rows0, acc.at[pl.ds(r0 + 128 * k, 128)])

        @pl.when(s < _NSUB - 1)
        def _():
            pltpu.sync_copy(rows0.at[pl.ds(0, 120)],
                            acc.at[pl.ds(r0 + 512, 120)])

        @pl.when(s == _NSUB - 1)
        def _():
            pltpu.sync_copy(rows0.at[pl.ds(0, 16)],
                            acc.at[pl.ds(r0 + 512, 16)])

        plsc.subcore_barrier()

        def gstart(k, slot):
            pltpu.async_copy(table.at[src_t.at[k]], rows[slot], sems[slot])

        def gwait(k, slot):
            pltpu.make_async_copy(table.at[src_t.at[k]], rows[slot],
                                  sems[slot]).wait()

        def scat(k, slot):
            pltpu.sync_copy(rows[slot], acc.at[dst_t.at[k]], add=True)

        gstart(0, 0)

        def body(j, carry):
            k = 2 * j
            gstart(k + 1, 1)
            gwait(k, 0)
            scat(k, 0)
            gstart(k + 2, 0)
            gwait(k + 1, 1)
            scat(k + 1, 1)
            return carry

        lax.fori_loop(0, _PAIRS, body, 0)
        gwait(_NBAT - 1, 0)
        scat(_NBAT - 1, 0)

        plsc.subcore_barrier()

        def chunk_out(off, size, buf):
            pltpu.sync_copy(acc.at[pl.ds(r0 + off, size)],
                            buf.at[pl.ds(0, size)])
            pltpu.sync_copy(buf.at[pl.ds(0, size)],
                            out.at[pl.ds(c * n + r0 + off, size)])

        for k in range(4):
            chunk_out(128 * k, 128, rows[k % 2])

        @pl.when(s < _NSUB - 1)
        def _():
            chunk_out(512, 120, rows0)

        @pl.when(s == _NSUB - 1)
        def _():
            chunk_out(512, 8, rows0)

    return seg


_SEG_SUM_CACHE = []


def _seg_sum(table_split, srcs, dsts, zeros):
    if not _SEG_SUM_CACHE:
        _SEG_SUM_CACHE.append(_make_seg_sum(_N, _DH))
    return _SEG_SUM_CACHE[0](table_split, srcs, dsts, zeros)


def _prep_edges(src, dst):
    s2 = jnp.pad(src.reshape(_NSUB, _EPT),
                 ((0, 0), (0, _NBAT * _BATCH - _EPT)))
    srcs = jnp.concatenate([s2, s2 + _N], axis=0).reshape(
        2 * _NSUB, _NBAT, _BATCH)
    d2 = jnp.pad(dst.reshape(_NSUB, _EPT),
                 ((0, 0), (0, _NBAT * _BATCH - _EPT)),
                 constant_values=_N)
    dsts = d2.reshape(_NSUB, _NBAT, _BATCH)
    return srcs, dsts


def _split(a):
    return jnp.concatenate([a[:, :_DH], a[:, _DH:]], axis=0)


_BR = 1000
_NBLK = _N // _BR


def _elu(y):
    return jnp.where(y > 0, y, jnp.exp(jnp.minimum(y, 0.0)) - 1.0)


def _lbe_body(has_agg, f):
    def body(*refs):
        if has_agg:
            (x_ref, pa_ref, pb_ref, w_ref, b_ref, g_ref, bb_ref,
             z_ref, y_acc, s1, s2) = refs
        else:
            (x_ref, w_ref, b_ref, g_ref, bb_ref, z_ref, y_acc, s1, s2) = refs
        p = pl.program_id(0)
        i = pl.program_id(1)

        @pl.when(p == 0)
        def _():
            t = x_ref[...]
            if has_agg:
                t = t + jnp.concatenate([pa_ref[...], pb_ref[...]], axis=1)
            y = jnp.dot(t, w_ref[...], preferred_element_type=jnp.float32) \
                + b_ref[...]
            y_acc[pl.ds(i * _BR, _BR), :] = y
            i1 = jnp.sum(y, axis=0, keepdims=True)
            i2 = jnp.sum(y * y, axis=0, keepdims=True)
            s1[0:1, :] = jnp.where(i == 0, i1, s1[0:1, :] + i1)
            s2[0:1, :] = jnp.where(i == 0, i2, s2[0:1, :] + i2)

        @pl.when(p == 1)
        def _():
            y = y_acc[pl.ds(i * _BR, _BR), :]
            m = s1[0:1, :] * (1.0 / _N)
            v = s2[0:1, :] * (1.0 / _N) - m * m
            z_ref[...] = _elu((y - m) * lax.rsqrt(v + 1e-5) * g_ref[...]
                              + bb_ref[...])

    return body


def _lbe(x, w, b, g, bb, p=None):
    k = x.shape[1]
    f = w.shape[1]
    has_agg = p is not None
    row = lambda pp, ii: (ii, 0)
    in_specs = [pl.BlockSpec((_BR, k), row)]
    args = [x]
    if has_agg:
        in_specs += [pl.BlockSpec((_BR, _DH), row),
                     pl.BlockSpec((_BR, _DH), lambda pp, ii: (ii + _NBLK, 0))]
        args += [p, p]
    in_specs += [pl.BlockSpec((k, f), lambda pp, ii: (0, 0))] + \
        [pl.BlockSpec((1, f), lambda pp, ii: (0, 0))] * 3
    args += [w, b.reshape(1, -1), g.reshape(1, -1), bb.reshape(1, -1)]
    return pl.pallas_call(
        _lbe_body(has_agg, f),
        grid=(2, _NBLK),
        in_specs=in_specs,
        out_specs=pl.BlockSpec((_BR, f), row),
        out_shape=jax.ShapeDtypeStruct((_N, f), jnp.float32),
        scratch_shapes=[pltpu.VMEM((_N, f), jnp.float32),
                        pltpu.VMEM((8, f), jnp.float32),
                        pltpu.VMEM((8, f), jnp.float32)],
    )(*args)


def _gin_body(f1, f2):
    def body(x_ref, pa_ref, pb_ref, cw_ref, cb_ref, cg_ref, cbb_ref,
             lw_ref, lb_ref, lg_ref, lbb_ref, h_ref, z_ref,
             yc_acc, y1_acc, c1, c2, l1, l2):
        p = pl.program_id(0)
        i = pl.program_id(1)
        rows = pl.ds(i * _BR, _BR)

        @pl.when(p == 0)
        def _():
            t = x_ref[...] + jnp.concatenate([pa_ref[...], pb_ref[...]],
                                             axis=1)
            yc = jnp.dot(t, cw_ref[...], preferred_element_type=jnp.float32) \
                + cb_ref[...]
            yc_acc[rows, :] = yc
            i1 = jnp.sum(yc, axis=0, keepdims=True)
            i2 = jnp.sum(yc * yc, axis=0, keepdims=True)
            c1[0:1, :] = jnp.where(i == 0, i1, c1[0:1, :] + i1)
            c2[0:1, :] = jnp.where(i == 0, i2, c2[0:1, :] + i2)

        @pl.when(p == 1)
        def _():
            yc = yc_acc[rows, :]
            m = c1[0:1, :] * (1.0 / _N)
            v = c2[0:1, :] * (1.0 / _N) - m * m
            h = _elu((yc - m) * lax.rsqrt(v + 1e-5) * cg_ref[...]
                     + cbb_ref[...])
            yc_acc[rows, :] = h
            y1 = jnp.dot(h, lw_ref[...], preferred_element_type=jnp.float32) \
                + lb_ref[...]
            y1_acc[rows, :] = y1
            i1 = jnp.sum(y1, axis=0, keepdims=True)
            i2 = jnp.sum(y1 * y1, axis=0, keepdims=True)
            l1[0:1, :] = jnp.where(i == 0, i1, l1[0:1, :] + i1)
            l2[0:1, :] = jnp.where(i == 0, i2, l2[0:1, :] + i2)

        @pl.when(p == 2)
        def _():
            h_ref[...] = yc_acc[rows, :]
            y1 = y1_acc[rows, :]
            m = l1[0:1, :] * (1.0 / _N)
            v = l2[0:1, :] * (1.0 / _N) - m * m
            z_ref[...] = _elu((y1 - m) * lax.rsqrt(v + 1e-5) * lg_ref[...]
                              + lbb_ref[...])

    return body


def _gin_block(x, p, cw, cb, cg, cbb, lw, lb, lg, lbb):
    k = x.shape[1]
    f1 = cw.shape[1]
    f2 = lw.shape[1]
    ph0 = lambda pp, ii: (jnp.where(pp == 0, ii, 0), 0)
    ph2 = lambda pp, ii: (jnp.where(pp == 2, ii, 0), 0)
    fixed = lambda pp, ii: (0, 0)
    r2 = lambda a: a.reshape(1, -1)
    in_specs = [
        pl.BlockSpec((_BR, k), ph0),
        pl.BlockSpec((_BR, _DH), ph0),
        pl.BlockSpec((_BR, _DH), lambda pp, ii: (jnp.where(pp == 0, ii, 0)
                                                 + _NBLK, 0)),
        pl.BlockSpec((k, f1), fixed), pl.BlockSpec((1, f1), fixed),
        pl.BlockSpec((1, f1), fixed), pl.BlockSpec((1, f1), fixed),
        pl.BlockSpec((f1, f2), fixed), pl.BlockSpec((1, f2), fixed),
        pl.BlockSpec((1, f2), fixed), pl.BlockSpec((1, f2), fixed),
    ]
    return pl.pallas_call(
        _gin_body(f1, f2),
        grid=(3, _NBLK),
        in_specs=in_specs,
        out_specs=[pl.BlockSpec((_BR, f1), ph2), pl.BlockSpec((_BR, f2), ph2)],
        out_shape=[jax.ShapeDtypeStruct((_N, f1), jnp.float32),
                   jax.ShapeDtypeStruct((_N, f2), jnp.float32)],
        scratch_shapes=[pltpu.VMEM((_N, f1), jnp.float32),
                        pltpu.VMEM((_N, f2), jnp.float32),
                        pltpu.VMEM((8, f1), jnp.float32),
                        pltpu.VMEM((8, f1), jnp.float32),
                        pltpu.VMEM((8, f2), jnp.float32),
                        pltpu.VMEM((8, f2), jnp.float32)],
    )(x, p, p, cw, r2(cb), r2(cg), r2(cbb), lw, r2(lb), r2(lg), r2(lbb))


def _tc_pool(z0_ref, z1_ref, z2_ref, lws_ref, batch_ref, out_ref, zz_ref, acc):
    i = pl.program_id(0)
    lws = lws_ref[...]
    z0 = z0_ref[...] * lws[0, 0]
    z1 = z1_ref[...] * lws[0, 1]
    z2 = z2_ref[...] * lws[0, 2]
    zz_ref[...] = z0 + z1 + z2
    big = jnp.concatenate([z0, z1, z2], axis=1)
    batch = batch_ref[...]
    ninf = jnp.float32(-jnp.inf)
    rows = []
    for g in range(_G):
        rows.append(jnp.max(jnp.where(batch == g, big, ninf), axis=0))
    blockmax = jnp.stack(rows)
    prev = jnp.where(i == 0, jnp.full((_G, 30), ninf), acc[...])
    acc[...] = jnp.maximum(prev, blockmax)

    @pl.when(i == _NBLK - 1)
    def _():
        oc = acc[...]
        out_ref[...] = oc[:, 0:10] + oc[:, 10:20] + oc[:, 20:30]


def _f32(*shape):
    return jax.ShapeDtypeStruct(shape, jnp.float32)


def kernel(x, edge_index, batch, lw,
           lin0_W, lin0_b, bn0_g, bn0_b,
           lin1_W, lin1_b, bn1_g, bn1_b,
           lin2_W, lin2_b, bn2_g, bn2_b,
           conv0_W, conv0_b, cbn0_g, cbn0_b,
           conv1_W, conv1_b, cbn1_g, cbn1_b):
    srcs, dsts = _prep_edges(edge_index[0], edge_index[1])
    zeros = jnp.zeros((_BATCH, _DH), jnp.float32)

    p1 = _seg_sum(_split(x), srcs, dsts, zeros)

    z0 = _lbe(x, lin0_W, lin0_b, bn0_g, bn0_b)

    h1, z1 = _gin_block(x, p1, conv0_W, conv0_b, cbn0_g, cbn0_b,
                        lin1_W, lin1_b, bn1_g, bn1_b)

    p2 = _seg_sum(_split(h1), srcs, dsts, zeros)

    h2, z2 = _gin_block(h1, p2, conv1_W, conv1_b, cbn1_g, cbn1_b,
                        lin2_W, lin2_b, bn2_g, bn2_b)

    rowspec = pl.BlockSpec((_BR, 10), lambda i: (i, 0))
    whole = lambda shp: pl.BlockSpec(shp, lambda i: (0, 0))
    out, zz = pl.pallas_call(
        _tc_pool,
        grid=(_NBLK,),
        in_specs=[rowspec, rowspec, rowspec, whole((1, 3)),
                  pl.BlockSpec((_BR, 1), lambda i: (i, 0))],
        out_specs=[whole((_G, 10)), rowspec],
        out_shape=[_f32(_G, 10), _f32(_N, 10)],
        scratch_shapes=[pltpu.VMEM((_G, 30), jnp.float32)],
    )(z0, z1, z2, lw.reshape(1, 3), batch.reshape(-1, 1))
    return (out, zz, h2)


# speedup vs baseline: 1.2166x; 1.0332x over previous
import functools

import jax
import jax.numpy as jnp
from jax import lax
from jax.experimental import pallas as pl
from jax.experimental.pallas import tpu as pltpu
from jax.experimental.pallas import tpu_sc as plsc

_N = 10000
_E = 320000
_G = 32
_D = 128

_NCORES = 2
_NSUB = 16
_DH = _D // 2
_EPT = _E // _NSUB
_BATCH = 128
_NBAT = (_EPT + _BATCH - 1) // _BATCH
_PAIRS = (_NBAT - 1) // 2
_NPAD = _N + 8
_RPT = 632
_RLAST = _N - (_NSUB - 1) * _RPT
_ZLAST = _NPAD - (_NSUB - 1) * _RPT


def _make_seg_sum(n, d):
    mesh = plsc.VectorSubcoreMesh(core_axis_name="c", subcore_axis_name="s")
    scratch = [
        pltpu.VMEM((_NBAT, _BATCH), jnp.int32),
        pltpu.VMEM((_NBAT, _BATCH), jnp.int32),
        pltpu.VMEM((_BATCH, d), jnp.float32),
        pltpu.VMEM((_BATCH, d), jnp.float32),
        pltpu.VMEM_SHARED((_NPAD, d), jnp.float32),
        pltpu.SemaphoreType.DMA,
        pltpu.SemaphoreType.DMA,
    ]

    @functools.partial(
        pl.kernel,
        out_type=jax.ShapeDtypeStruct((2 * n, d), jnp.float32),
        mesh=mesh,
        scratch_types=scratch,
        compiler_params=pltpu.CompilerParams(use_tc_tiling_on_sc=False),
    )
    def seg(table, srcs, dsts, zeros, out,
            src_t, dst_t, rows0, rows1, acc, sem0, sem1):
        c = lax.axis_index("c")
        s = lax.axis_index("s")
        r0 = s * _RPT
        rows = (rows0, rows1)
        sems = (sem0, sem1)

        pltpu.sync_copy(srcs.at[c * _NSUB + s], src_t)
        pltpu.sync_copy(dsts.at[s], dst_t)

        pltpu.sync_copy(zeros, rows0)
        for k in range(4):
            pltpu.sync_copy(rows0, acc.at[pl.ds(r0 + 128 * k, 128)])

        @pl.when(s < _NSUB - 1)
        def _():
            pltpu.sync_copy(rows0.at[pl.ds(0, 120)],
                            acc.at[pl.ds(r0 + 512, 120)])

        @pl.when(s == _NSUB - 1)
        def _():
            pltpu.sync_copy(rows0.at[pl.ds(0, 16)],
                            acc.at[pl.ds(r0 + 512, 16)])

        plsc.subcore_barrier()

        def gstart(k, slot):
            pltpu.async_copy(table.at[src_t.at[k]], rows[slot], sems[slot])

        def gwait(k, slot):
            pltpu.make_async_copy(table.at[src_t.at[k]], rows[slot],
                                  sems[slot]).wait()

        def scat(k, slot):
            pltpu.sync_copy(rows[slot], acc.at[dst_t.at[k]], add=True)

        gstart(0, 0)

        def body(j, carry):
            k = 2 * j
            gstart(k + 1, 1)
            gwait(k, 0)
            scat(k, 0)
            gstart(k + 2, 0)
            gwait(k + 1, 1)
            scat(k + 1, 1)
            return carry

        lax.fori_loop(0, _PAIRS, body, 0)
        gwait(_NBAT - 1, 0)
        scat(_NBAT - 1, 0)

        plsc.subcore_barrier()

        def chunk_out(off, size, buf):
            pltpu.sync_copy(acc.at[pl.ds(r0 + off, size)],
                            buf.at[pl.ds(0, size)])
            pltpu.sync_copy(buf.at[pl.ds(0, size)],
                            out.at[pl.ds(c * n + r0 + off, size)])

        for k in range(4):
            chunk_out(128 * k, 128, rows[k % 2])

        @pl.when(s < _NSUB - 1)
        def _():
            chunk_out(512, 120, rows0)

        @pl.when(s == _NSUB - 1)
        def _():
            chunk_out(512, 8, rows0)

    return seg


_SEG_SUM_CACHE = []


def _seg_sum(table_split, srcs, dsts, zeros):
    if not _SEG_SUM_CACHE:
        _SEG_SUM_CACHE.append(_make_seg_sum(_N, _DH))
    return _SEG_SUM_CACHE[0](table_split, srcs, dsts, zeros)


def _prep_edges(src, dst):
    s2 = jnp.pad(src.reshape(_NSUB, _EPT),
                 ((0, 0), (0, _NBAT * _BATCH - _EPT)))
    srcs = jnp.concatenate([s2, s2 + _N], axis=0).reshape(
        2 * _NSUB, _NBAT, _BATCH)
    d2 = jnp.pad(dst.reshape(_NSUB, _EPT),
                 ((0, 0), (0, _NBAT * _BATCH - _EPT)),
                 constant_values=_N)
    dsts = d2.reshape(_NSUB, _NBAT, _BATCH)
    return srcs, dsts


def _split(a):
    return jnp.concatenate([a[:, :_DH], a[:, _DH:]], axis=0)


_BR = 2000
_NBLK = _N // _BR


def _elu(y):
    return jnp.where(y > 0, y, jnp.exp(jnp.minimum(y, 0.0)) - 1.0)


def _norm_elu(y, s1, s2, g, b):
    m = s1[0:1, :] * (1.0 / _N)
    v = s2[0:1, :] * (1.0 / _N) - m * m
    return _elu((y - m) * lax.rsqrt(v + 1e-5) * g + b)


def _acc_stats(s1, s2, i, y):
    i1 = jnp.sum(y, axis=0, keepdims=True)
    i2 = jnp.sum(y * y, axis=0, keepdims=True)
    s1[0:1, :] = jnp.where(i == 0, i1, s1[0:1, :] + i1)
    s2[0:1, :] = jnp.where(i == 0, i2, s2[0:1, :] + i2)


def _dot(a, w, b):
    return jnp.dot(a, w, preferred_element_type=jnp.float32) + b


def _tc_a_body(x_ref, pa_ref, pb_ref, w0_ref, b0_ref, g0_ref, bb0_ref,
               cw_ref, cb_ref, cg_ref, cbb_ref,
               w1_ref, b1_ref, g1_ref, bb1_ref,
               h_ref, z0_ref, z1_ref,
               yc_acc, y0_acc, y1_acc, c1, c2, a1, a2, l1, l2):
    p = pl.program_id(0)
    i = pl.program_id(1)
    rows = pl.ds(i * _BR, _BR)

    @pl.when(p == 0)
    def _():
        x = x_ref[...]
        y0 = _dot(x, w0_ref[...], b0_ref[...])
        y0_acc[rows, :] = y0
        _acc_stats(a1, a2, i, y0)
        t = x + jnp.concatenate([pa_ref[...], pb_ref[...]], axis=1)
        yc = _dot(t, cw_ref[...], cb_ref[...])
        yc_acc[rows, :] = yc
        _acc_stats(c1, c2, i, yc)

    @pl.when(p == 1)
    def _():
        h = _norm_elu(yc_acc[rows, :], c1, c2, cg_ref[...], cbb_ref[...])
        yc_acc[rows, :] = h
        y1 = _dot(h, w1_ref[...], b1_ref[...])
        y1_acc[rows, :] = y1
        _acc_stats(l1, l2, i, y1)

    @pl.when(p == 2)
    def _():
        h_ref[...] = yc_acc[rows, :]
        z0_ref[...] = _norm_elu(y0_acc[rows, :], a1, a2,
                                g0_ref[...], bb0_ref[...])
        z1_ref[...] = _norm_elu(y1_acc[rows, :], l1, l2,
                                g1_ref[...], bb1_ref[...])


def _tc_b_body(h1_ref, pa_ref, pb_ref, cw_ref, cb_ref, cg_ref, cbb_ref,
               w2_ref, b2_ref, g2_ref, bb2_ref,
               z0_ref, z1_ref, lws_ref, batch_ref,
               h2_ref, zz_ref, out_ref,
               yc_acc, y2_acc, c1, c2, l1, l2, pacc):
    p = pl.program_id(0)
    i = pl.program_id(1)
    rows = pl.ds(i * _BR, _BR)

    @pl.when(p == 0)
    def _():
        t = h1_ref[...] + jnp.concatenate([pa_ref[...], pb_ref[...]], axis=1)
        yc = _dot(t, cw_ref[...], cb_ref[...])
        yc_acc[rows, :] = yc
        _acc_stats(c1, c2, i, yc)

    @pl.when(p == 1)
    def _():
        h2 = _norm_elu(yc_acc[rows, :], c1, c2, cg_ref[...], cbb_ref[...])
        yc_acc[rows, :] = h2
        y2 = _dot(h2, w2_ref[...], b2_ref[...])
        y2_acc[rows, :] = y2
        _acc_stats(l1, l2, i, y2)

    @pl.when(p == 2)
    def _():
        h2_ref[...] = yc_acc[rows, :]
        lws = lws_ref[...]
        z0 = z0_ref[...] * lws[0, 0]
        z1 = z1_ref[...] * lws[0, 1]
        z2 = _norm_elu(y2_acc[rows, :], l1, l2,
                       g2_ref[...], bb2_ref[...]) * lws[0, 2]
        zz_ref[...] = z0 + z1 + z2
        big = jnp.concatenate([z0, z1, z2], axis=1)
        batch = batch_ref[...]
        ninf = jnp.float32(-jnp.inf)
        rows_mx = []
        for g in range(_G):
            rows_mx.append(jnp.max(jnp.where(batch == g, big, ninf), axis=0))
        blockmax = jnp.stack(rows_mx)
        prev = jnp.where(i == 0, jnp.full((_G, 30), ninf), pacc[...])
        pacc[...] = jnp.maximum(prev, blockmax)

        @pl.when(i == _NBLK - 1)
        def _():
            oc = pacc[...]
            out_ref[...] = oc[:, 0:10] + oc[:, 10:20] + oc[:, 20:30]


def _f32(*shape):
    return jax.ShapeDtypeStruct(shape, jnp.float32)


def _vm(*shape):
    return pltpu.VMEM(shape, jnp.float32)


_ROW = lambda pp, ii: (jnp.where(pp == 0, ii, 0), 0)
_ROWB = lambda pp, ii: (jnp.where(pp == 0, ii, 0) + _NBLK, 0)
_ROW2 = lambda pp, ii: (jnp.where(pp == 2, ii, 0), 0)
_FIX = lambda pp, ii: (0, 0)


def kernel(x, edge_index, batch, lw,
           lin0_W, lin0_b, bn0_g, bn0_b,
           lin1_W, lin1_b, bn1_g, bn1_b,
           lin2_W, lin2_b, bn2_g, bn2_b,
           conv0_W, conv0_b, cbn0_g, cbn0_b,
           conv1_W, conv1_b, cbn1_g, cbn1_b):
    srcs, dsts = _prep_edges(edge_index[0], edge_index[1])
    zeros = jnp.zeros((_BATCH, _DH), jnp.float32)
    r2 = lambda a: a.reshape(1, -1)
    wspec = lambda k, f: [pl.BlockSpec((k, f), _FIX)] + \
        [pl.BlockSpec((1, f), _FIX)] * 3

    p1 = _seg_sum(_split(x), srcs, dsts, zeros)

    h1, z0, z1 = pl.pallas_call(
        _tc_a_body,
        grid=(3, _NBLK),
        in_specs=[pl.BlockSpec((_BR, _D), _ROW),
                  pl.BlockSpec((_BR, _DH), _ROW),
                  pl.BlockSpec((_BR, _DH), _ROWB)]
        + wspec(_D, 10) + wspec(_D, _D) + wspec(_D, 10),
        out_specs=[pl.BlockSpec((_BR, _D), _ROW2),
                   pl.BlockSpec((_BR, 10), _ROW2),
                   pl.BlockSpec((_BR, 10), _ROW2)],
        out_shape=[_f32(_N, _D), _f32(_N, 10), _f32(_N, 10)],
        scratch_shapes=[_vm(_N, _D), _vm(_N, 10), _vm(_N, 10),
                        _vm(8, _D), _vm(8, _D), _vm(8, 10), _vm(8, 10),
                        _vm(8, 10), _vm(8, 10)],
    )(x, p1, p1,
      lin0_W, r2(lin0_b), r2(bn0_g), r2(bn0_b),
      conv0_W, r2(conv0_b), r2(cbn0_g), r2(cbn0_b),
      lin1_W, r2(lin1_b), r2(bn1_g), r2(bn1_b))

    p2 = _seg_sum(_split(h1), srcs, dsts, zeros)

    h2, zz, out = pl.pallas_call(
        _tc_b_body,
        grid=(3, _NBLK),
        in_specs=[pl.BlockSpec((_BR, _D), _ROW),
                  pl.BlockSpec((_BR, _DH), _ROW),
                  pl.BlockSpec((_BR, _DH), _ROWB)]
        + wspec(_D, _D) + wspec(_D, 10)
        + [pl.BlockSpec((_BR, 10), _ROW2),
           pl.BlockSpec((_BR, 10), _ROW2),
           pl.BlockSpec((1, 3), _FIX),
           pl.BlockSpec((_BR, 1), _ROW2)],
        out_specs=[pl.BlockSpec((_BR, _D), _ROW2),
                   pl.BlockSpec((_BR, 10), _ROW2),
                   pl.BlockSpec((_G, 10), _FIX)],
        out_shape=[_f32(_N, _D), _f32(_N, 10), _f32(_G, 10)],
        scratch_shapes=[_vm(_N, _D), _vm(_N, 10),
                        _vm(8, _D), _vm(8, _D), _vm(8, 10), _vm(8, 10),
                        _vm(_G, 30)],
    )(h1, p2, p2,
      conv1_W, r2(conv1_b), r2(cbn1_g), r2(cbn1_b),
      lin2_W, r2(lin2_b), r2(bn2_g), r2(bn2_b),
      z0, z1, lw.reshape(1, 3), batch.reshape(-1, 1))

    return (out, zz, h2)
